# baseline (speedup 1.0000x reference)
import functools

import numpy as np
import jax
import jax.numpy as jnp
from jax import lax
from jax.experimental import pallas as pl
from jax.experimental.pallas import tpu as pltpu
from jax.experimental.pallas import tpu_sc as plsc

B, L, D, M = 4096, 50, 64, 128

NC, NS = 2, 16
NW = NC * NS
SEQ_PER_W = (B * L) // NW
CHUNK = 320
N_CHUNKS = SEQ_PER_W // CHUNK
BID_PER_W = B // NW

_E_EXPAND = np.repeat(np.eye(D, dtype=np.float32), D, axis=1)

BB = 256
NB = B // BB


def _sc_gather_body(src_iid_h, seq_h, src_uidT_h, uid_h, tgt_iidT_h, iid_h,
                    ufea_h, uemb_h, iemb_h,
                    idx_v, rows_v, col_v, sel_v, sem, csem):
    wid = lax.axis_index("s") * NC + lax.axis_index("c")
    base = wid * SEQ_PER_W

    def chunk(ci, carry):
        off = base + ci * CHUNK
        pltpu.sync_copy(seq_h.at[pl.ds(off, CHUNK)], idx_v)

        def issue(t, c):
            iv = idx_v[pl.ds(t * 16, 16)]
            for k in range(16):
                r = iv[k]
                pltpu.async_copy(
                    src_iid_h.at[pl.ds(r, 1)],
                    rows_v.at[pl.ds(t * 16 + k, 1)], sem)
            return c

        lax.fori_loop(0, CHUNK // 16, issue, 0)
        pltpu.make_async_copy(
            src_iid_h.at[pl.ds(0, CHUNK)], rows_v, sem).wait()
        pltpu.sync_copy(rows_v, ufea_h.at[pl.ds(off, CHUNK)])
        return carry

    lax.fori_loop(0, N_CHUNKS, chunk, 0)

    def small_gather(tabT_h, sidx_h, out_h):
        b0 = wid * BID_PER_W
        pltpu.sync_copy(sidx_h.at[pl.ds(b0, BID_PER_W)],
                        idx_v.at[pl.ds(0, BID_PER_W)])

        def one(t, carry):
            iv = idx_v[pl.ds(t * 16, 16)]
            cv = jax.lax.bitwise_and(iv, 127)
            tv = jax.lax.shift_right_logical(iv, 7) * 128
            for quarter in range(4):
                for k4 in range(4):
                    t0 = pl.multiple_of(tv[quarter * 4 + k4], 128)
                    for dt in range(8):
                        pltpu.async_copy(
                            tabT_h.at[pl.ds(dt * 8, 8), pl.ds(t0, 128)],
                            col_v.at[k4, pl.ds(dt * 8, 8)], csem)
                pltpu.make_async_copy(tabT_h.at[:, pl.ds(0, 128 * 4)],
                                      col_v, csem).wait()
                for k4 in range(4):
                    k = quarter * 4 + k4
                    jv = jnp.full((16,), t * 16 + k, jnp.int32)
                    kv = jnp.full((16,), k4, jnp.int32)
                    lv = jnp.full((16,), 0, jnp.int32) + cv[k]
                    for du in range(D // 16):
                        dv = jax.lax.iota(jnp.int32, 16) + du * 16
                        val = plsc.load_gather(col_v, [kv, dv, lv])
                        plsc.store_scatter(sel_v, [jv, dv], val)
            return carry

        lax.fori_loop(0, BID_PER_W // 16, one, 0)
        pltpu.sync_copy(sel_v, out_h.at[pl.ds(b0, BID_PER_W)])

    small_gather(src_uidT_h, uid_h, uemb_h)
    small_gather(tgt_iidT_h, iid_h, iemb_h)


def _sc_gather(src_iid, seq_flat, src_uidT, uid_idx, tgt_iidT, iid_idx):
    return pl.kernel(
        _sc_gather_body,
        out_type=[
            jax.ShapeDtypeStruct((B * L, D), jnp.float32),
            jax.ShapeDtypeStruct((B, D), jnp.float32),
            jax.ShapeDtypeStruct((B, D), jnp.float32),
        ],
        mesh=plsc.VectorSubcoreMesh(core_axis_name="c", subcore_axis_name="s"),
        scratch_types=[
            pltpu.VMEM((CHUNK,), jnp.int32),
            pltpu.VMEM((CHUNK, D), jnp.float32),
            pltpu.VMEM((4, D, 128), jnp.float32),
            pltpu.VMEM((BID_PER_W, D), jnp.float32),
            pltpu.SemaphoreType.DMA,
            pltpu.SemaphoreType.DMA,
        ],
        compiler_params=pltpu.CompilerParams(needs_layout_passes=False),
    )(src_iid, seq_flat, src_uidT, uid_idx, tgt_iidT, iid_idx)


def _tc_body(seq_ref, ufea_ref, uemb_ref, iemb_ref,
             W1_ref, b1_ref, w2_ref, dW1_ref, db1_ref, dW2_ref, db2_ref,
             tw_ref, E_ref, out_ref):
    W1 = W1_ref[...]
    b1 = b1_ref[...]
    w2 = w2_ref[...]

    cols = []
    for l in range(L):
        ul = ufea_ref[:, l, :]
        h = jnp.maximum(jnp.dot(ul, W1, preferred_element_type=jnp.float32)
                        + b1, 0.0)
        cols.append(jnp.sum(h * w2, axis=1, keepdims=True))
    ek = jnp.concatenate(cols, axis=1)

    mask = seq_ref[...] == 0
    t = jnp.where(mask, ek - 1e8, ek)
    t = t - jnp.max(t, axis=1, keepdims=True)
    e = jnp.exp(t)
    att = e / jnp.sum(e, axis=1, keepdims=True)

    his = jnp.zeros((BB, D), jnp.float32)
    for l in range(L):
        his = his + att[:, l:l + 1] * ufea_ref[:, l, :]

    g = jnp.maximum(jnp.dot(his, dW1_ref[...],
                            preferred_element_type=jnp.float32) + db1_ref[...],
                    0.0)
    dec = jnp.dot(g, dW2_ref[...],
                  preferred_element_type=jnp.float32) + db2_ref[...]

    q = iemb_ref[...] * tw_ref[...]
    qt = jnp.concatenate([q] * D, axis=1)
    ur = jnp.dot(uemb_ref[...], E_ref[...],
                 preferred_element_type=jnp.float32)
    out_ref[...] = jnp.sum(dec * qt * ur, axis=1, keepdims=True)


def _tc_forward(seq, ufea3, uemb, iemb,
                eK_W1, eK_b1, eK_w2, dec_W1, dec_b1, dec_W2, dec_b2, tgt_w,
                interpret=False):
    out = pl.pallas_call(
        _tc_body,
        grid=(NB,),
        in_specs=[
            pl.BlockSpec((BB, L), lambda i: (i, 0)),
            pl.BlockSpec((BB, L, D), lambda i: (i, 0, 0)),
            pl.BlockSpec((BB, D), lambda i: (i, 0)),
            pl.BlockSpec((BB, D), lambda i: (i, 0)),
            pl.BlockSpec((D, D), lambda i: (0, 0)),
            pl.BlockSpec((1, D), lambda i: (0, 0)),
            pl.BlockSpec((1, D), lambda i: (0, 0)),
            pl.BlockSpec((D, M), lambda i: (0, 0)),
            pl.BlockSpec((1, M), lambda i: (0, 0)),
            pl.BlockSpec((M, D * D), lambda i: (0, 0)),
            pl.BlockSpec((1, D * D), lambda i: (0, 0)),
            pl.BlockSpec((1, D), lambda i: (0, 0)),
            pl.BlockSpec((D, D * D), lambda i: (0, 0)),
        ],
        out_specs=pl.BlockSpec((BB, 1), lambda i: (i, 0)),
        out_shape=jax.ShapeDtypeStruct((B, 1), jnp.float32),
        compiler_params=pltpu.CompilerParams(
            dimension_semantics=("arbitrary",)),
        interpret=interpret,
    )(seq, ufea3, uemb, iemb,
      eK_W1, eK_b1.reshape(1, D), eK_w2.reshape(1, D),
      dec_W1, dec_b1.reshape(1, M), dec_W2, dec_b2.reshape(1, D * D),
      tgt_w.reshape(1, D), jnp.asarray(_E_EXPAND))
    return out[:, 0]


def kernel(x, src_uid, src_iid, tgt_iid, eK_W1, eK_b1, eK_w2,
           dec_W1, dec_b1, dec_W2, dec_b2, tgt_w):
    seq = x[:, 2:]
    seq_flat = seq.reshape(-1)
    ids = jnp.arange(1000008, dtype=jnp.int32)
    src_iid_rm = jnp.take(src_iid, ids, axis=0, mode='clip')
    ufea_flat, uemb, iemb = _sc_gather(
        src_iid_rm, seq_flat, src_uid.T, x[:, 0], tgt_iid.T, x[:, 1])
    return _tc_forward(
        seq, ufea_flat.reshape(B, L, D), uemb, iemb,
        eK_W1, eK_b1, eK_w2, dec_W1, dec_b1, dec_W2, dec_b2, tgt_w)


# speedup vs baseline: 1.6954x; 1.6954x over previous
import functools

import numpy as np
import jax
import jax.numpy as jnp
from jax import lax
from jax.experimental import pallas as pl
from jax.experimental.pallas import tpu as pltpu
from jax.experimental.pallas import tpu_sc as plsc

B, L, D, M = 4096, 50, 64, 128

NC, NS = 2, 16
NW = NC * NS
SEQ_PER_W = (B * L) // NW
CHUNK = 320
N_CHUNKS = SEQ_PER_W // CHUNK
BID_PER_W = B // NW

_E_EXPAND = np.repeat(np.eye(D, dtype=np.float32), D, axis=1)

BB = 256
NB = B // BB


def _sc_bulk_body(src_iid_h, seq_h, ufea_h, idx_v, rows_v, sem):
    wid = lax.axis_index("s") * NC + lax.axis_index("c")
    base = wid * SEQ_PER_W

    def chunk(ci, carry):
        off = base + ci * CHUNK
        pltpu.sync_copy(seq_h.at[pl.ds(off, CHUNK)], idx_v)

        def issue(t, c):
            iv = idx_v[pl.ds(t * 16, 16)]
            for k in range(16):
                r = iv[k]
                pltpu.async_copy(
                    src_iid_h.at[pl.ds(r, 1)],
                    rows_v.at[pl.ds(t * 16 + k, 1)], sem)
            return c

        lax.fori_loop(0, CHUNK // 16, issue, 0)
        pltpu.make_async_copy(
            src_iid_h.at[pl.ds(0, CHUNK)], rows_v, sem).wait()
        pltpu.sync_copy(rows_v, ufea_h.at[pl.ds(off, CHUNK)])
        return carry

    lax.fori_loop(0, N_CHUNKS, chunk, 0)


def _sc_small_body(src_uidT_h, uid_h, tgt_iidT_h, iid_h,
                   uemb_h, iemb_h,
                   idx_v, col_v, sel_v, csem):
    wid = lax.axis_index("s") * NC + lax.axis_index("c")

    def small_gather(tabT_h, sidx_h, out_h):
        b0 = wid * BID_PER_W
        pltpu.sync_copy(sidx_h.at[pl.ds(b0, BID_PER_W)],
                        idx_v.at[pl.ds(0, BID_PER_W)])

        def one(t, carry):
            iv = idx_v[pl.ds(t * 16, 16)]
            cv = jax.lax.bitwise_and(iv, 127)
            tv = jax.lax.shift_right_logical(iv, 7) * 128
            for quarter in range(4):
                for k4 in range(4):
                    t0 = pl.multiple_of(tv[quarter * 4 + k4], 128)
                    for dt in range(8):
                        pltpu.async_copy(
                            tabT_h.at[pl.ds(dt * 8, 8), pl.ds(t0, 128)],
                            col_v.at[k4, pl.ds(dt * 8, 8)], csem)
                pltpu.make_async_copy(tabT_h.at[:, pl.ds(0, 128 * 4)],
                                      col_v, csem).wait()
                for k4 in range(4):
                    k = quarter * 4 + k4
                    jv = jnp.full((16,), t * 16 + k, jnp.int32)
                    kv = jnp.full((16,), k4, jnp.int32)
                    lv = jnp.full((16,), 0, jnp.int32) + cv[k]
                    for du in range(D // 16):
                        dv = jax.lax.iota(jnp.int32, 16) + du * 16
                        val = plsc.load_gather(col_v, [kv, dv, lv])
                        plsc.store_scatter(sel_v, [jv, dv], val)
            return carry

        lax.fori_loop(0, BID_PER_W // 16, one, 0)
        pltpu.sync_copy(sel_v, out_h.at[pl.ds(b0, BID_PER_W)])

    small_gather(src_uidT_h, uid_h, uemb_h)
    small_gather(tgt_iidT_h, iid_h, iemb_h)


def _sc_bulk(src_iid, seq_flat):
    return pl.kernel(
        _sc_bulk_body,
        out_type=[jax.ShapeDtypeStruct((B * L, D), jnp.float32)],
        mesh=plsc.VectorSubcoreMesh(core_axis_name="c", subcore_axis_name="s"),
        scratch_types=[
            pltpu.VMEM((CHUNK,), jnp.int32),
            pltpu.VMEM((CHUNK, D), jnp.float32),
            pltpu.SemaphoreType.DMA,
        ],
        compiler_params=pltpu.CompilerParams(needs_layout_passes=False),
    )(src_iid, seq_flat)


def _sc_small(src_uidT, uid_idx, tgt_iidT, iid_idx):
    return pl.kernel(
        _sc_small_body,
        out_type=[
            jax.ShapeDtypeStruct((B, D), jnp.float32),
            jax.ShapeDtypeStruct((B, D), jnp.float32),
        ],
        mesh=plsc.VectorSubcoreMesh(core_axis_name="c", subcore_axis_name="s"),
        scratch_types=[
            pltpu.VMEM((BID_PER_W,), jnp.int32),
            pltpu.VMEM((4, D, 128), jnp.float32),
            pltpu.VMEM((BID_PER_W, D), jnp.float32),
            pltpu.SemaphoreType.DMA,
        ],
        compiler_params=pltpu.CompilerParams(needs_layout_passes=False),
    )(src_uidT, uid_idx, tgt_iidT, iid_idx)


def _tc_body(seq_ref, ufea_ref, uemb_ref, iemb_ref,
             W1_ref, b1_ref, w2_ref, dW1_ref, db1_ref, dW2_ref, db2_ref,
             tw_ref, E_ref, out_ref):
    W1 = W1_ref[...]
    b1 = b1_ref[...]
    w2 = w2_ref[...]

    cols = []
    for l in range(L):
        ul = ufea_ref[:, l, :]
        h = jnp.maximum(jnp.dot(ul, W1, preferred_element_type=jnp.float32)
                        + b1, 0.0)
        cols.append(jnp.sum(h * w2, axis=1, keepdims=True))
    ek = jnp.concatenate(cols, axis=1)

    mask = seq_ref[...] == 0
    t = jnp.where(mask, ek - 1e8, ek)
    t = t - jnp.max(t, axis=1, keepdims=True)
    e = jnp.exp(t)
    att = e / jnp.sum(e, axis=1, keepdims=True)

    his = jnp.zeros((BB, D), jnp.float32)
    for l in range(L):
        his = his + att[:, l:l + 1] * ufea_ref[:, l, :]

    g = jnp.maximum(jnp.dot(his, dW1_ref[...],
                            preferred_element_type=jnp.float32) + db1_ref[...],
                    0.0)
    dec = jnp.dot(g, dW2_ref[...],
                  preferred_element_type=jnp.float32) + db2_ref[...]

    q = iemb_ref[...] * tw_ref[...]
    qt = jnp.concatenate([q] * D, axis=1)
    ur = jnp.dot(uemb_ref[...], E_ref[...],
                 preferred_element_type=jnp.float32)
    out_ref[...] = jnp.sum(dec * qt * ur, axis=1, keepdims=True)


def _tc_forward(seq, ufea3, uemb, iemb,
                eK_W1, eK_b1, eK_w2, dec_W1, dec_b1, dec_W2, dec_b2, tgt_w,
                interpret=False):
    out = pl.pallas_call(
        _tc_body,
        grid=(NB,),
        in_specs=[
            pl.BlockSpec((BB, L), lambda i: (i, 0)),
            pl.BlockSpec((BB, L, D), lambda i: (i, 0, 0)),
            pl.BlockSpec((BB, D), lambda i: (i, 0)),
            pl.BlockSpec((BB, D), lambda i: (i, 0)),
            pl.BlockSpec((D, D), lambda i: (0, 0)),
            pl.BlockSpec((1, D), lambda i: (0, 0)),
            pl.BlockSpec((1, D), lambda i: (0, 0)),
            pl.BlockSpec((D, M), lambda i: (0, 0)),
            pl.BlockSpec((1, M), lambda i: (0, 0)),
            pl.BlockSpec((M, D * D), lambda i: (0, 0)),
            pl.BlockSpec((1, D * D), lambda i: (0, 0)),
            pl.BlockSpec((1, D), lambda i: (0, 0)),
            pl.BlockSpec((D, D * D), lambda i: (0, 0)),
        ],
        out_specs=pl.BlockSpec((BB, 1), lambda i: (i, 0)),
        out_shape=jax.ShapeDtypeStruct((B, 1), jnp.float32),
        compiler_params=pltpu.CompilerParams(
            dimension_semantics=("arbitrary",)),
        interpret=interpret,
    )(seq, ufea3, uemb, iemb,
      eK_W1, eK_b1.reshape(1, D), eK_w2.reshape(1, D),
      dec_W1, dec_b1.reshape(1, M), dec_W2, dec_b2.reshape(1, D * D),
      tgt_w.reshape(1, D), jnp.asarray(_E_EXPAND))
    return out[:, 0]


def kernel(x, src_uid, src_iid, tgt_iid, eK_W1, eK_b1, eK_w2,
           dec_W1, dec_b1, dec_W2, dec_b2, tgt_w):
    seq = x[:, 2:]
    seq_flat = seq.reshape(-1)
    uemb, iemb = _sc_small(src_uid.T, x[:, 0], tgt_iid.T, x[:, 1])
    (ufea_flat,) = _sc_bulk(src_iid, seq_flat)
    return _tc_forward(
        seq, ufea_flat.reshape(B, L, D), uemb, iemb,
        eK_W1, eK_b1, eK_w2, dec_W1, dec_b1, dec_W2, dec_b2, tgt_w)


# speedup vs baseline: 1.7212x; 1.0152x over previous
import functools

import numpy as np
import jax
import jax.numpy as jnp
from jax import lax
from jax.experimental import pallas as pl
from jax.experimental.pallas import tpu as pltpu
from jax.experimental.pallas import tpu_sc as plsc

B, L, D, M = 4096, 50, 64, 128

NC, NS = 2, 16
NW = NC * NS
SEQ_PER_W = (B * L) // NW
CHUNK = 640
N_CHUNKS = SEQ_PER_W // CHUNK
BID_PER_W = B // NW

_E_EXPAND = np.repeat(np.eye(D, dtype=np.float32), D, axis=1)

BB = 256
NB = B // BB


def _sc_bulk_body(src_iid_h, seq_h, ufea_h, idx_v, rows_v, sem):
    wid = lax.axis_index("s") * NC + lax.axis_index("c")
    base = wid * SEQ_PER_W

    def chunk(ci, carry):
        off = base + ci * CHUNK
        pltpu.sync_copy(seq_h.at[pl.ds(off, CHUNK)], idx_v)

        def issue(t, c):
            iv = idx_v[pl.ds(t * 16, 16)]
            for k in range(16):
                r = iv[k]
                pltpu.async_copy(
                    src_iid_h.at[pl.ds(r, 1)],
                    rows_v.at[pl.ds(t * 16 + k, 1)], sem)
            return c

        lax.fori_loop(0, CHUNK // 16, issue, 0)
        pltpu.make_async_copy(
            src_iid_h.at[pl.ds(0, CHUNK)], rows_v, sem).wait()
        pltpu.sync_copy(rows_v, ufea_h.at[pl.ds(off, CHUNK)])
        return carry

    lax.fori_loop(0, N_CHUNKS, chunk, 0)


def _sc_small_body(src_uidT_h, uid_h, tgt_iidT_h, iid_h,
                   uemb_h, iemb_h,
                   idx_v, col_v, sel_v, csem):
    wid = lax.axis_index("s") * NC + lax.axis_index("c")

    def small_gather(tabT_h, sidx_h, out_h):
        b0 = wid * BID_PER_W
        pltpu.sync_copy(sidx_h.at[pl.ds(b0, BID_PER_W)],
                        idx_v.at[pl.ds(0, BID_PER_W)])

        def one(t, carry):
            iv = idx_v[pl.ds(t * 16, 16)]
            cv = jax.lax.bitwise_and(iv, 127)
            tv = jax.lax.shift_right_logical(iv, 7) * 128
            for quarter in range(4):
                for k4 in range(4):
                    t0 = pl.multiple_of(tv[quarter * 4 + k4], 128)
                    for dt in range(8):
                        pltpu.async_copy(
                            tabT_h.at[pl.ds(dt * 8, 8), pl.ds(t0, 128)],
                            col_v.at[k4, pl.ds(dt * 8, 8)], csem)
                pltpu.make_async_copy(tabT_h.at[:, pl.ds(0, 128 * 4)],
                                      col_v, csem).wait()
                for k4 in range(4):
                    k = quarter * 4 + k4
                    jv = jnp.full((16,), t * 16 + k, jnp.int32)
                    kv = jnp.full((16,), k4, jnp.int32)
                    lv = jnp.full((16,), 0, jnp.int32) + cv[k]
                    for du in range(D // 16):
                        dv = jax.lax.iota(jnp.int32, 16) + du * 16
                        val = plsc.load_gather(col_v, [kv, dv, lv])
                        plsc.store_scatter(sel_v, [jv, dv], val)
            return carry

        lax.fori_loop(0, BID_PER_W // 16, one, 0)
        pltpu.sync_copy(sel_v, out_h.at[pl.ds(b0, BID_PER_W)])

    small_gather(src_uidT_h, uid_h, uemb_h)
    small_gather(tgt_iidT_h, iid_h, iemb_h)


def _sc_bulk(src_iid, seq_flat):
    return pl.kernel(
        _sc_bulk_body,
        out_type=[jax.ShapeDtypeStruct((B * L, D), jnp.float32)],
        mesh=plsc.VectorSubcoreMesh(core_axis_name="c", subcore_axis_name="s"),
        scratch_types=[
            pltpu.VMEM((CHUNK,), jnp.int32),
            pltpu.VMEM((CHUNK, D), jnp.float32),
            pltpu.SemaphoreType.DMA,
        ],
        compiler_params=pltpu.CompilerParams(needs_layout_passes=False),
    )(src_iid, seq_flat)


def _sc_small(src_uidT, uid_idx, tgt_iidT, iid_idx):
    return pl.kernel(
        _sc_small_body,
        out_type=[
            jax.ShapeDtypeStruct((B, D), jnp.float32),
            jax.ShapeDtypeStruct((B, D), jnp.float32),
        ],
        mesh=plsc.VectorSubcoreMesh(core_axis_name="c", subcore_axis_name="s"),
        scratch_types=[
            pltpu.VMEM((BID_PER_W,), jnp.int32),
            pltpu.VMEM((4, D, 128), jnp.float32),
            pltpu.VMEM((BID_PER_W, D), jnp.float32),
            pltpu.SemaphoreType.DMA,
        ],
        compiler_params=pltpu.CompilerParams(needs_layout_passes=False),
    )(src_uidT, uid_idx, tgt_iidT, iid_idx)


def _tc_body(seq_ref, ufea_ref, uemb_ref, iemb_ref,
             W1_ref, b1_ref, w2_ref, dW1_ref, db1_ref, dW2_ref, db2_ref,
             tw_ref, E_ref, out_ref):
    W1 = W1_ref[...]
    b1 = b1_ref[...]
    w2 = w2_ref[...]

    cols = []
    for l in range(L):
        ul = ufea_ref[:, l, :]
        h = jnp.maximum(jnp.dot(ul, W1, preferred_element_type=jnp.float32)
                        + b1, 0.0)
        cols.append(jnp.sum(h * w2, axis=1, keepdims=True))
    ek = jnp.concatenate(cols, axis=1)

    mask = seq_ref[...] == 0
    t = jnp.where(mask, ek - 1e8, ek)
    t = t - jnp.max(t, axis=1, keepdims=True)
    e = jnp.exp(t)
    att = e / jnp.sum(e, axis=1, keepdims=True)

    his = jnp.zeros((BB, D), jnp.float32)
    for l in range(L):
        his = his + att[:, l:l + 1] * ufea_ref[:, l, :]

    g = jnp.maximum(jnp.dot(his, dW1_ref[...],
                            preferred_element_type=jnp.float32) + db1_ref[...],
                    0.0)
    dec = jnp.dot(g, dW2_ref[...],
                  preferred_element_type=jnp.float32) + db2_ref[...]

    q = iemb_ref[...] * tw_ref[...]
    qt = jnp.concatenate([q] * D, axis=1)
    ur = jnp.dot(uemb_ref[...], E_ref[...],
                 preferred_element_type=jnp.float32)
    out_ref[...] = jnp.sum(dec * qt * ur, axis=1, keepdims=True)


def _tc_forward(seq, ufea3, uemb, iemb,
                eK_W1, eK_b1, eK_w2, dec_W1, dec_b1, dec_W2, dec_b2, tgt_w,
                interpret=False):
    out = pl.pallas_call(
        _tc_body,
        grid=(NB,),
        in_specs=[
            pl.BlockSpec((BB, L), lambda i: (i, 0)),
            pl.BlockSpec((BB, L, D), lambda i: (i, 0, 0)),
            pl.BlockSpec((BB, D), lambda i: (i, 0)),
            pl.BlockSpec((BB, D), lambda i: (i, 0)),
            pl.BlockSpec((D, D), lambda i: (0, 0)),
            pl.BlockSpec((1, D), lambda i: (0, 0)),
            pl.BlockSpec((1, D), lambda i: (0, 0)),
            pl.BlockSpec((D, M), lambda i: (0, 0)),
            pl.BlockSpec((1, M), lambda i: (0, 0)),
            pl.BlockSpec((M, D * D), lambda i: (0, 0)),
            pl.BlockSpec((1, D * D), lambda i: (0, 0)),
            pl.BlockSpec((1, D), lambda i: (0, 0)),
            pl.BlockSpec((D, D * D), lambda i: (0, 0)),
        ],
        out_specs=pl.BlockSpec((BB, 1), lambda i: (i, 0)),
        out_shape=jax.ShapeDtypeStruct((B, 1), jnp.float32),
        compiler_params=pltpu.CompilerParams(
            dimension_semantics=("arbitrary",)),
        interpret=interpret,
    )(seq, ufea3, uemb, iemb,
      eK_W1, eK_b1.reshape(1, D), eK_w2.reshape(1, D),
      dec_W1, dec_b1.reshape(1, M), dec_W2, dec_b2.reshape(1, D * D),
      tgt_w.reshape(1, D), jnp.asarray(_E_EXPAND))
    return out[:, 0]


def kernel(x, src_uid, src_iid, tgt_iid, eK_W1, eK_b1, eK_w2,
           dec_W1, dec_b1, dec_W2, dec_b2, tgt_w):
    seq = x[:, 2:]
    seq_flat = seq.reshape(-1)
    uemb, iemb = _sc_small(src_uid.T, x[:, 0], tgt_iid.T, x[:, 1])
    (ufea_flat,) = _sc_bulk(src_iid, seq_flat)
    return _tc_forward(
        seq, ufea_flat.reshape(B, L, D), uemb, iemb,
        eK_W1, eK_b1, eK_w2, dec_W1, dec_b1, dec_W2, dec_b2, tgt_w)
